# Initial kernel scaffold; baseline (speedup 1.0000x reference)
import jax
import jax.numpy as jnp
from jax.experimental import pallas as pl


def kernel(x, w_qkv, b_qkv, w_o, b_o, ln1_g, ln1_b, ln2_g, ln2_b, gate_w, gate_b, w1, b1, w2, b2):
    raise NotImplementedError("write your pallas kernel here")


# speedup vs baseline: 1.1175x; 1.1175x over previous
import functools

import jax
import jax.numpy as jnp
from jax import lax
from jax.experimental import pallas as pl
from jax.experimental.pallas import tpu as pltpu
from jax.experimental.pallas import tpu_sc as plsc

S, B, D, H, FF, E = 2048, 1, 768, 12, 3072, 8
T = S * B
C = (2 * T) // E
HD = D // H
NW = 32
TPW = T // NW
EC = E * C
EC_PAD = EC + NW
QB = 512
RB = 512
FB = FF // 2

_INTERPRET = False


def _qkv_body(x_ref, w_ref, b_ref, o_ref):
    o_ref[...] = (
        jnp.dot(x_ref[...], w_ref[...], preferred_element_type=jnp.float32)
        + b_ref[...]
    )


def _qkv_proj(x2, wqkv_t, b_qkv):
    return pl.pallas_call(
        _qkv_body,
        grid=(T // QB, 3),
        in_specs=[
            pl.BlockSpec((QB, D), lambda i, j: (i, 0)),
            pl.BlockSpec((D, D), lambda i, j: (0, j)),
            pl.BlockSpec((1, D), lambda i, j: (0, j)),
        ],
        out_specs=pl.BlockSpec((QB, D), lambda i, j: (i, j)),
        out_shape=jax.ShapeDtypeStruct((T, 3 * D), jnp.float32),
        interpret=_INTERPRET,
    )(x2, wqkv_t, b_qkv.reshape(1, 3 * D))


def _attn_body(q_ref, k_ref, v_ref, o_ref):
    q = q_ref[...].reshape(QB, HD)
    k = k_ref[...].reshape(T, HD)
    s = lax.dot_general(q, k, (((1,), (1,)), ((), ())),
                        preferred_element_type=jnp.float32) * (1.0 / 8.0)
    m = jnp.max(s, axis=-1, keepdims=True)
    p = jnp.exp(s - m)
    l = jnp.sum(p, axis=-1, keepdims=True)
    o = jnp.dot(p, v_ref[...].reshape(T, HD),
                preferred_element_type=jnp.float32) / l
    o_ref[...] = o.reshape(QB, 1, 1, HD)


def _attention(qkv4):
    return pl.pallas_call(
        _attn_body,
        grid=(H, T // QB),
        in_specs=[
            pl.BlockSpec((QB, 1, 1, HD), lambda h, i: (i, h, 0, 0)),
            pl.BlockSpec((T, 1, 1, HD), lambda h, i: (0, H + h, 0, 0)),
            pl.BlockSpec((T, 1, 1, HD), lambda h, i: (0, 2 * H + h, 0, 0)),
        ],
        out_specs=pl.BlockSpec((QB, 1, 1, HD), lambda h, i: (i, h, 0, 0)),
        out_shape=jax.ShapeDtypeStruct((T, H, 1, HD), jnp.float32),
        interpret=_INTERPRET,
    )(qkv4, qkv4, qkv4)


def _post_body(attn_ref, wo_ref, bo_ref, x_ref, g_ref, b_ref, gw_ref, gb_ref,
               h1_ref, dd_ref, dc_ref, keep_ref, oh_scr):
    o = (jnp.dot(attn_ref[...], wo_ref[...], preferred_element_type=jnp.float32)
         + bo_ref[...] + x_ref[...])
    m = jnp.mean(o, axis=-1, keepdims=True)
    zc = o - m
    v = jnp.mean(zc * zc, axis=-1, keepdims=True)
    h1 = zc * lax.rsqrt(v + 1e-5) * g_ref[...] + b_ref[...]
    h1_ref[...] = h1

    logits = (jnp.dot(h1, gw_ref[...], preferred_element_type=jnp.float32)
              + gb_ref[...])
    lm = jnp.max(logits, axis=-1, keepdims=True)
    lane = lax.broadcasted_iota(jnp.int32, (T, 128), 1)
    eidx = jnp.min(jnp.where(logits == lm, lane, 128), axis=-1, keepdims=True)
    dd_ref[...] = eidx
    oh_scr[...] = (lane == eidx).astype(jnp.float32)

    rr = lax.broadcasted_iota(jnp.int32, (RB, RB), 0)
    cc = lax.broadcasted_iota(jnp.int32, (RB, RB), 1)
    tril = (rr > cc).astype(jnp.float32)

    def blk(i, cnts):
        start = pl.multiple_of(i * RB, RB)
        ohb = oh_scr[pl.ds(start, RB), :]
        posb = jnp.dot(tril, ohb, preferred_element_type=jnp.float32) + cnts
        pie = jnp.sum(posb * ohb, axis=-1, keepdims=True)
        eb = dd_ref[pl.ds(start, RB), :]
        keep = pie < float(C)
        slot = jnp.minimum(pie.astype(jnp.int32), C - 1)
        dslot = eb * C + slot
        tok = start + lax.broadcasted_iota(jnp.int32, (RB, 1), 0)
        dd_ref[pl.ds(start, RB), :] = jnp.where(keep, dslot, EC + tok // TPW)
        dc_ref[pl.ds(start, RB), :] = jnp.where(keep, dslot, 0)
        keep_ref[pl.ds(start, RB), :] = keep.astype(jnp.float32)
        return cnts + jnp.sum(ohb, axis=0, keepdims=True)

    lax.fori_loop(0, T // RB, blk, jnp.zeros((1, 128), jnp.float32))


def _post_attn(attn, wo_t, b_o, x2, ln1_g, ln1_b, gw_pad, gb_pad):
    return pl.pallas_call(
        _post_body,
        out_shape=(
            jax.ShapeDtypeStruct((T, D), jnp.float32),
            jax.ShapeDtypeStruct((T, 1), jnp.int32),
            jax.ShapeDtypeStruct((T, 1), jnp.int32),
            jax.ShapeDtypeStruct((T, 1), jnp.float32),
        ),
        scratch_shapes=[pltpu.VMEM((T, 128), jnp.float32)],
        interpret=_INTERPRET,
    )(attn, wo_t, b_o.reshape(1, D), x2, ln1_g.reshape(1, D),
      ln1_b.reshape(1, D), gw_pad, gb_pad)


def _sc_worker_id():
    return lax.axis_index("s") * 2 + lax.axis_index("c")


def _sc_dispatch_body(h1_hbm, dd_hbm, disp_hbm, idx_v, rows_v, sem):
    base = _sc_worker_id() * TPW
    pltpu.sync_copy(dd_hbm.at[pl.ds(base, TPW)], idx_v)
    pltpu.sync_copy(h1_hbm.at[pl.ds(base, TPW)], rows_v)
    pltpu.async_copy(rows_v, disp_hbm.at[idx_v], sem).wait()


def _sc_combine_body(o_hbm, dc_hbm, y_hbm, idx_v, rows_v, sem):
    base = _sc_worker_id() * TPW
    pltpu.sync_copy(dc_hbm.at[pl.ds(base, TPW)], idx_v)
    pltpu.async_copy(o_hbm.at[idx_v], rows_v, sem).wait()
    pltpu.sync_copy(rows_v, y_hbm.at[pl.ds(base, TPW)])


def _sc_call(body, out_rows):
    mesh = plsc.VectorSubcoreMesh(core_axis_name="c", subcore_axis_name="s",
                                  num_cores=2, num_subcores=16)
    return pl.kernel(
        body,
        out_type=jax.ShapeDtypeStruct((out_rows, D), jnp.float32),
        mesh=mesh,
        scratch_types=[
            pltpu.VMEM((TPW,), jnp.int32),
            pltpu.VMEM((TPW, D), jnp.float32),
            pltpu.SemaphoreType.DMA,
        ],
        interpret=_INTERPRET,
    )


def _erf(x):
    s = jnp.sign(x)
    a = jnp.abs(x)
    t = 1.0 / (1.0 + 0.3275911 * a)
    poly = t * (0.254829592 + t * (-0.284496736 + t * (1.421413741
               + t * (-1.453152027 + t * 1.061405429))))
    return s * (1.0 - poly * jnp.exp(-a * a))


def _gelu_exact(x):
    return 0.5 * x * (1.0 + _erf(x * 0.7071067811865476))


def _moe_body(d_ref, w1_ref, b1_ref, w2_ref, b2_ref, o_ref):
    fb = pl.program_id(1)
    h = (jnp.dot(d_ref[...], w1_ref[0], preferred_element_type=jnp.float32)
         + b1_ref[0])
    h = _gelu_exact(h)
    part = jnp.dot(h, w2_ref[0], preferred_element_type=jnp.float32)

    @pl.when(fb == 0)
    def _():
        o_ref[...] = part + b2_ref[0]

    @pl.when(fb != 0)
    def _():
        o_ref[...] += part


def _expert_mlp(disp, w1t, b1, w2t, b2):
    return pl.pallas_call(
        _moe_body,
        grid=(E, FF // FB),
        in_specs=[
            pl.BlockSpec((C, D), lambda e, f: (e, 0)),
            pl.BlockSpec((1, D, FB), lambda e, f: (e, 0, f)),
            pl.BlockSpec((1, 1, FB), lambda e, f: (e, 0, f)),
            pl.BlockSpec((1, FB, D), lambda e, f: (e, f, 0)),
            pl.BlockSpec((1, 1, D), lambda e, f: (e, 0, 0)),
        ],
        out_specs=pl.BlockSpec((C, D), lambda e, f: (e, 0)),
        out_shape=jax.ShapeDtypeStruct((EC, D), jnp.float32),
        interpret=_INTERPRET,
    )(disp, w1t, b1.reshape(E, 1, FF), w2t, b2.reshape(E, 1, D))


def _final_body(y_ref, k_ref, h1_ref, g_ref, b_ref, o_ref):
    z = h1_ref[...] + y_ref[...] * k_ref[...]
    m = jnp.mean(z, axis=-1, keepdims=True)
    zc = z - m
    v = jnp.mean(zc * zc, axis=-1, keepdims=True)
    o_ref[...] = zc * lax.rsqrt(v + 1e-5) * g_ref[...] + b_ref[...]


def _final(y, keepf, h1, ln2_g, ln2_b):
    return pl.pallas_call(
        _final_body,
        grid=(T // QB,),
        in_specs=[
            pl.BlockSpec((QB, D), lambda i: (i, 0)),
            pl.BlockSpec((QB, 1), lambda i: (i, 0)),
            pl.BlockSpec((QB, D), lambda i: (i, 0)),
            pl.BlockSpec((1, D), lambda i: (0, 0)),
            pl.BlockSpec((1, D), lambda i: (0, 0)),
        ],
        out_specs=pl.BlockSpec((QB, D), lambda i: (i, 0)),
        out_shape=jax.ShapeDtypeStruct((T, D), jnp.float32),
        interpret=_INTERPRET,
    )(y, keepf, h1, ln2_g.reshape(1, D), ln2_b.reshape(1, D))


def kernel(x, w_qkv, b_qkv, w_o, b_o, ln1_g, ln1_b, ln2_g, ln2_b,
           gate_w, gate_b, w1, b1, w2, b2):
    f32 = jnp.float32
    x2 = x.reshape(T, D)
    wqkv_t = w_qkv.T
    wo_t = w_o.T
    gw_pad = jnp.pad(gate_w.T, ((0, 0), (0, 128 - E)))
    gb_pad = jnp.concatenate(
        [gate_b, jnp.full((128 - E,), -1e30, f32)]).reshape(1, 128)
    w1t = w1.transpose(0, 2, 1)
    w2t = w2.transpose(0, 2, 1)

    qkv = _qkv_proj(x2, wqkv_t, b_qkv)
    attn4 = _attention(qkv.reshape(T, 3 * H, 1, HD))
    attn = attn4.reshape(T, D)
    h1, dd, dc, keepf = _post_attn(attn, wo_t, b_o, x2, ln1_g, ln1_b,
                                   gw_pad, gb_pad)

    disp = _sc_call(_sc_dispatch_body, EC_PAD)(h1, dd.reshape(T))
    moe = _expert_mlp(disp, w1t, b1, w2t, b2)
    y = _sc_call(_sc_combine_body, T)(moe, dc.reshape(T))

    out = _final(y, keepf, h1, ln2_g, ln2_b)
    return out.reshape(S, B, D)


# speedup vs baseline: 1.2602x; 1.1278x over previous
import functools

import jax
import jax.numpy as jnp
from jax import lax
from jax.experimental import pallas as pl
from jax.experimental.pallas import tpu as pltpu
from jax.experimental.pallas import tpu_sc as plsc

S, B, D, H, FF, E = 2048, 1, 768, 12, 3072, 8
T = S * B
C = (2 * T) // E
HD = D // H
NW = 32
TPW = T // NW
EC = E * C
EC_PAD = EC + NW
QB = 512
AQB = 1024
RB = 512
FB = FF // 2

_INTERPRET = False


def _dot_nt(a, b):
    return lax.dot_general(a.astype(jnp.bfloat16), b.astype(jnp.bfloat16),
                           (((1,), (1,)), ((), ())),
                           preferred_element_type=jnp.float32)


def _dot_nn(a, b):
    return jnp.dot(a.astype(jnp.bfloat16), b.astype(jnp.bfloat16),
                   preferred_element_type=jnp.float32)


def _qkv_body(x_ref, w_ref, b_ref, o_ref):
    o_ref[...] = _dot_nt(x_ref[...], w_ref[...]) + b_ref[...]


def _qkv_proj(x2, w_qkv, b_qkv):
    return pl.pallas_call(
        _qkv_body,
        grid=(T // QB, 3),
        in_specs=[
            pl.BlockSpec((QB, D), lambda i, j: (i, 0)),
            pl.BlockSpec((D, D), lambda i, j: (j, 0)),
            pl.BlockSpec((1, D), lambda i, j: (0, j)),
        ],
        out_specs=pl.BlockSpec((QB, D), lambda i, j: (i, j)),
        out_shape=jax.ShapeDtypeStruct((T, 3 * D), jnp.float32),
        interpret=_INTERPRET,
    )(x2, w_qkv, b_qkv.reshape(1, 3 * D))


def _attn_body(q_ref, k_ref, v_ref, o_ref):
    q = q_ref[...].reshape(AQB, HD)
    k = k_ref[...].reshape(T, HD)
    s = _dot_nt(q, k) * (1.0 / 8.0)
    m = jnp.max(s, axis=-1, keepdims=True)
    p = jnp.exp(s - m)
    a = p / jnp.sum(p, axis=-1, keepdims=True)
    o = _dot_nn(a, v_ref[...].reshape(T, HD))
    o_ref[...] = o.reshape(AQB, 1, 1, HD)


def _attention(qkv4):
    return pl.pallas_call(
        _attn_body,
        grid=(H, T // AQB),
        in_specs=[
            pl.BlockSpec((AQB, 1, 1, HD), lambda h, i: (i, h, 0, 0)),
            pl.BlockSpec((T, 1, 1, HD), lambda h, i: (0, H + h, 0, 0)),
            pl.BlockSpec((T, 1, 1, HD), lambda h, i: (0, 2 * H + h, 0, 0)),
        ],
        out_specs=pl.BlockSpec((AQB, 1, 1, HD), lambda h, i: (i, h, 0, 0)),
        out_shape=jax.ShapeDtypeStruct((T, H, 1, HD), jnp.float32),
        interpret=_INTERPRET,
    )(qkv4, qkv4, qkv4)


def _post_body(attn_ref, wo_ref, bo_ref, x_ref, g_ref, b_ref, gw_ref, gb_ref,
               h1_ref, dd_ref, dc_ref, keep_ref, oh_scr):
    o = _dot_nt(attn_ref[...], wo_ref[...]) + bo_ref[...] + x_ref[...]
    m = jnp.mean(o, axis=-1, keepdims=True)
    zc = o - m
    v = jnp.mean(zc * zc, axis=-1, keepdims=True)
    h1 = zc / jnp.sqrt(v + 1e-5) * g_ref[...] + b_ref[...]
    h1_ref[...] = h1

    logits = _dot_nn(h1, gw_ref[...]) + gb_ref[...]
    lm = jnp.max(logits, axis=-1, keepdims=True)
    lane = lax.broadcasted_iota(jnp.int32, (T, 128), 1)
    eidx = jnp.min(jnp.where(logits == lm, lane, 128), axis=-1, keepdims=True)
    dd_ref[...] = eidx
    oh8 = (lax.broadcasted_iota(jnp.int32, (T, E), 1) == eidx)
    oh_scr[...] = oh8.astype(jnp.float32)

    rr = lax.broadcasted_iota(jnp.int32, (RB, RB), 0)
    cc = lax.broadcasted_iota(jnp.int32, (RB, RB), 1)
    tril = (rr > cc).astype(jnp.float32)

    def blk(i, cnts):
        start = pl.multiple_of(i * RB, RB)
        ohb = oh_scr[pl.ds(start, RB), :]
        posb = jnp.dot(tril, ohb, preferred_element_type=jnp.float32) + cnts
        pie = jnp.sum(posb * ohb, axis=-1, keepdims=True)
        eb = dd_ref[pl.ds(start, RB), :]
        keep = pie < float(C)
        slot = jnp.minimum(pie.astype(jnp.int32), C - 1)
        dslot = eb * C + slot
        tok = start + lax.broadcasted_iota(jnp.int32, (RB, 1), 0)
        dd_ref[pl.ds(start, RB), :] = jnp.where(keep, dslot, EC + tok // TPW)
        dc_ref[pl.ds(start, RB), :] = jnp.where(keep, dslot, 0)
        keep_ref[pl.ds(start, RB), :] = keep.astype(jnp.float32)
        return cnts + jnp.sum(ohb, axis=0, keepdims=True)

    lax.fori_loop(0, T // RB, blk, jnp.zeros((1, E), jnp.float32))


def _post_attn(attn, w_o, b_o, x2, ln1_g, ln1_b, gw_pad, gb_pad):
    return pl.pallas_call(
        _post_body,
        out_shape=(
            jax.ShapeDtypeStruct((T, D), jnp.float32),
            jax.ShapeDtypeStruct((T, 1), jnp.int32),
            jax.ShapeDtypeStruct((T, 1), jnp.int32),
            jax.ShapeDtypeStruct((T, 1), jnp.float32),
        ),
        scratch_shapes=[pltpu.VMEM((T, E), jnp.float32)],
        interpret=_INTERPRET,
    )(attn, w_o, b_o.reshape(1, D), x2, ln1_g.reshape(1, D),
      ln1_b.reshape(1, D), gw_pad, gb_pad)


def _sc_worker_id():
    return lax.axis_index("s") * 2 + lax.axis_index("c")


def _sc_dispatch_body(h1_hbm, dd_hbm, disp_hbm, idx_v, rows_v, sem):
    base = _sc_worker_id() * TPW
    pltpu.sync_copy(dd_hbm.at[pl.ds(base, TPW)], idx_v)
    pltpu.sync_copy(h1_hbm.at[pl.ds(base, TPW)], rows_v)
    pltpu.async_copy(rows_v, disp_hbm.at[idx_v], sem).wait()


def _sc_combine_body(o_hbm, dc_hbm, y_hbm, idx_v, rows_v, sem):
    base = _sc_worker_id() * TPW
    pltpu.sync_copy(dc_hbm.at[pl.ds(base, TPW)], idx_v)
    pltpu.async_copy(o_hbm.at[idx_v], rows_v, sem).wait()
    pltpu.sync_copy(rows_v, y_hbm.at[pl.ds(base, TPW)])


def _sc_call(body, out_rows):
    mesh = plsc.VectorSubcoreMesh(core_axis_name="c", subcore_axis_name="s",
                                  num_cores=2, num_subcores=16)
    return pl.kernel(
        body,
        out_type=jax.ShapeDtypeStruct((out_rows, D), jnp.float32),
        mesh=mesh,
        scratch_types=[
            pltpu.VMEM((TPW,), jnp.int32),
            pltpu.VMEM((TPW, D), jnp.float32),
            pltpu.SemaphoreType.DMA,
        ],
        interpret=_INTERPRET,
    )


def _erf(x):
    s = jnp.sign(x)
    a = jnp.abs(x)
    t = 1.0 / (1.0 + 0.3275911 * a)
    poly = t * (0.254829592 + t * (-0.284496736 + t * (1.421413741
               + t * (-1.453152027 + t * 1.061405429))))
    return s * (1.0 - poly * jnp.exp(-a * a))


def _gelu_exact(x):
    return 0.5 * x * (1.0 + _erf(x * 0.7071067811865476))


def _moe_body(d_ref, w1_ref, b1_ref, w2_ref, b2_ref, o_ref):
    fb = pl.program_id(1)
    h = _dot_nt(d_ref[...], w1_ref[0]) + b1_ref[0]
    h = _gelu_exact(h)
    part = _dot_nt(h, w2_ref[0])

    @pl.when(fb == 0)
    def _():
        o_ref[...] = part + b2_ref[0]

    @pl.when(fb != 0)
    def _():
        o_ref[...] += part


def _expert_mlp(disp, w1, b1, w2, b2):
    return pl.pallas_call(
        _moe_body,
        grid=(E, FF // FB),
        in_specs=[
            pl.BlockSpec((C, D), lambda e, f: (e, 0)),
            pl.BlockSpec((1, FB, D), lambda e, f: (e, f, 0)),
            pl.BlockSpec((1, 1, FB), lambda e, f: (e, 0, f)),
            pl.BlockSpec((1, D, FB), lambda e, f: (e, 0, f)),
            pl.BlockSpec((1, 1, D), lambda e, f: (e, 0, 0)),
        ],
        out_specs=pl.BlockSpec((C, D), lambda e, f: (e, 0)),
        out_shape=jax.ShapeDtypeStruct((EC, D), jnp.float32),
        interpret=_INTERPRET,
    )(disp, w1, b1.reshape(E, 1, FF), w2, b2.reshape(E, 1, D))


def _final_body(y_ref, k_ref, h1_ref, g_ref, b_ref, o_ref):
    z = h1_ref[...] + y_ref[...] * k_ref[...]
    m = jnp.mean(z, axis=-1, keepdims=True)
    zc = z - m
    v = jnp.mean(zc * zc, axis=-1, keepdims=True)
    o_ref[...] = zc / jnp.sqrt(v + 1e-5) * g_ref[...] + b_ref[...]


def _final(y, keepf, h1, ln2_g, ln2_b):
    return pl.pallas_call(
        _final_body,
        grid=(T // QB,),
        in_specs=[
            pl.BlockSpec((QB, D), lambda i: (i, 0)),
            pl.BlockSpec((QB, 1), lambda i: (i, 0)),
            pl.BlockSpec((QB, D), lambda i: (i, 0)),
            pl.BlockSpec((1, D), lambda i: (0, 0)),
            pl.BlockSpec((1, D), lambda i: (0, 0)),
        ],
        out_specs=pl.BlockSpec((QB, D), lambda i: (i, 0)),
        out_shape=jax.ShapeDtypeStruct((T, D), jnp.float32),
        interpret=_INTERPRET,
    )(y, keepf, h1, ln2_g.reshape(1, D), ln2_b.reshape(1, D))


def kernel(x, w_qkv, b_qkv, w_o, b_o, ln1_g, ln1_b, ln2_g, ln2_b,
           gate_w, gate_b, w1, b1, w2, b2):
    x2 = x.reshape(T, D)
    gw_pad = jnp.pad(gate_w.T, ((0, 0), (0, 128 - E)))
    gb_pad = jnp.concatenate(
        [gate_b, jnp.full((128 - E,), -1e30, jnp.float32)]).reshape(1, 128)
    qkv = _qkv_proj(x2, w_qkv, b_qkv)
    attn4 = _attention(qkv.reshape(T, 3 * H, 1, HD))
    attn = attn4.reshape(T, D)
    h1, dd, dc, keepf = _post_attn(attn, w_o, b_o, x2, ln1_g, ln1_b,
                                   gw_pad, gb_pad)

    disp = _sc_call(_sc_dispatch_body, EC_PAD)(h1, dd.reshape(T))
    moe = _expert_mlp(disp, w1, b1, w2, b2)
    y = _sc_call(_sc_combine_body, T)(moe, dc.reshape(T))

    out = _final(y, keepf, h1, ln2_g, ln2_b)
    return out.reshape(S, B, D)
